# baseline (device time: 373531 ns/iter reference)
import functools

import jax
import jax.numpy as jnp
from jax import lax
from jax.experimental import pallas as pl
from jax.experimental.pallas import tpu as pltpu

N = 8
SQ = 1024
HP = 8
DH = 128
HD = HP * DH
KVW = 1152
KV0 = 1024
KV1 = 128
CH = SQ // N
SCALE = 0.08838834764831843


def _body(x_ref, wq_ref, k_hbm, v_hbm, wo_ref, out_ref,
          q16, kbuf, vbuf, ctx16, acc, red,
          arbuf, agbuf,
          kv_send, kv_recv, loc_sem,
          ar_send, ar_recv, ag_send, ag_recv):
    my = lax.axis_index("i")

    def _kv_rdma(src_rows, src_cols, dst_rows, send_slot, recv_slot, dest):
        return pltpu.make_async_remote_copy(
            src_ref=k_hbm.at[src_rows, src_cols],
            dst_ref=kbuf.at[dst_rows, :],
            send_sem=kv_send.at[send_slot],
            recv_sem=kv_recv.at[recv_slot],
            device_id=(dest,),
            device_id_type=pl.DeviceIdType.MESH,
        )

    def _kv_rdma_v(src_rows, src_cols, dst_rows, send_slot, recv_slot, dest):
        return pltpu.make_async_remote_copy(
            src_ref=v_hbm.at[src_rows, src_cols],
            dst_ref=vbuf.at[dst_rows, :],
            send_sem=kv_send.at[send_slot],
            recv_sem=kv_recv.at[recv_slot],
            device_id=(dest,),
            device_id_type=pl.DeviceIdType.MESH,
        )

    r0 = pl.ds(0, KV0)
    r1 = pl.ds(KV0, KV1)

    @pl.when(my == 0)
    def _():
        for j in range(1, N):
            cols = pl.ds(j * HD, HD)
            _kv_rdma(pl.ds(0, KV0), cols, r0, j - 1, 0, j).start()
            _kv_rdma_v(pl.ds(0, KV0), cols, r0, 7 + j - 1, 1, j).start()
        ck = pltpu.make_async_copy(
            k_hbm.at[pl.ds(0, KV0), pl.ds(0, HD)], kbuf.at[r0, :], loc_sem.at[0])
        cv = pltpu.make_async_copy(
            v_hbm.at[pl.ds(0, KV0), pl.ds(0, HD)], vbuf.at[r0, :], loc_sem.at[1])
        ck.start(); cv.start(); ck.wait(); cv.wait()

    @pl.when(my == 1)
    def _():
        for t, j in enumerate([0, 2, 3, 4, 5, 6, 7]):
            cols = pl.ds(j * HD, HD)
            _kv_rdma(pl.ds(0, KV1), cols, r1, t, 2, j).start()
            _kv_rdma_v(pl.ds(0, KV1), cols, r1, 7 + t, 3, j).start()
        ck = pltpu.make_async_copy(
            k_hbm.at[pl.ds(0, KV1), pl.ds(HD, HD)], kbuf.at[r1, :], loc_sem.at[0])
        cv = pltpu.make_async_copy(
            v_hbm.at[pl.ds(0, KV1), pl.ds(HD, HD)], vbuf.at[r1, :], loc_sem.at[1])
        ck.start(); cv.start(); ck.wait(); cv.wait()

    q = jnp.dot(x_ref[...], wq_ref[...], preferred_element_type=jnp.float32)
    q16[...] = (q * SCALE).astype(jnp.bfloat16)

    @pl.when(my != 0)
    def _():
        _kv_rdma(pl.ds(0, KV0), pl.ds(0, HD), r0, 15, 0, 0).wait_recv()
        _kv_rdma_v(pl.ds(0, KV0), pl.ds(0, HD), r0, 15, 1, 0).wait_recv()

    @pl.when(my != 1)
    def _():
        _kv_rdma(pl.ds(0, KV1), pl.ds(0, HD), r1, 15, 2, 1).wait_recv()
        _kv_rdma_v(pl.ds(0, KV1), pl.ds(0, HD), r1, 15, 3, 1).wait_recv()

    qi = lax.broadcasted_iota(jnp.int32, (SQ, KVW), 0)
    ki = lax.broadcasted_iota(jnp.int32, (SQ, KVW), 1)
    mask = jnp.abs(qi - ki) <= 128
    for h in range(HP):
        c = pl.ds(h * DH, DH)
        s = lax.dot_general(
            q16[:, c], kbuf[:, c],
            (((1,), (1,)), ((), ())), preferred_element_type=jnp.float32)
        s = jnp.where(mask, s, -1e9)
        m = jnp.max(s, axis=1, keepdims=True)
        w = jnp.exp(s - m)
        w = w / jnp.sum(w, axis=1, keepdims=True)
        ctx = lax.dot_general(
            w.astype(jnp.bfloat16), vbuf[:, c],
            (((1,), (0,)), ((), ())), preferred_element_type=jnp.float32)
        ctx16[:, c] = ctx.astype(jnp.bfloat16)

    acc[...] = jnp.dot(ctx16[...], wo_ref[...],
                       preferred_element_type=jnp.float32)

    for o in range(1, N):
        dest = lax.rem(my + o, N)
        pltpu.make_async_remote_copy(
            src_ref=acc.at[pl.ds(dest * CH, CH), :],
            dst_ref=arbuf.at[o - 1],
            send_sem=ar_send.at[o - 1],
            recv_sem=ar_recv.at[o - 1],
            device_id=(dest,),
            device_id_type=pl.DeviceIdType.MESH,
        ).start()
    for o in range(1, N):
        pltpu.make_async_remote_copy(
            src_ref=arbuf.at[o - 1], dst_ref=arbuf.at[o - 1],
            send_sem=ar_send.at[o - 1], recv_sem=ar_recv.at[o - 1],
            device_id=(0,), device_id_type=pl.DeviceIdType.MESH,
        ).wait_recv()

    r = acc[pl.ds(my * CH, CH), :]
    for t in range(N - 1):
        r = r + arbuf[t]
    red[...] = r
    out_ref[pl.ds(my * CH, CH), :] = r

    for o in range(1, N):
        dest = lax.rem(my + o, N)
        pltpu.make_async_remote_copy(
            src_ref=red,
            dst_ref=agbuf.at[o - 1],
            send_sem=ag_send.at[o - 1],
            recv_sem=ag_recv.at[o - 1],
            device_id=(dest,),
            device_id_type=pl.DeviceIdType.MESH,
        ).start()
    for o in range(1, N):
        pltpu.make_async_remote_copy(
            src_ref=agbuf.at[o - 1], dst_ref=agbuf.at[o - 1],
            send_sem=ag_send.at[o - 1], recv_sem=ag_recv.at[o - 1],
            device_id=(0,), device_id_type=pl.DeviceIdType.MESH,
        ).wait_recv()
        src_pos = lax.rem(my - o + N, N)
        out_ref[pl.ds(src_pos * CH, CH), :] = agbuf[o - 1]

    @pl.when(my == 0)
    def _():
        for j in range(1, N):
            cols = pl.ds(j * HD, HD)
            _kv_rdma(pl.ds(0, KV0), cols, r0, j - 1, 0, j).wait_send()
            _kv_rdma_v(pl.ds(0, KV0), cols, r0, 7 + j - 1, 1, j).wait_send()

    @pl.when(my == 1)
    def _():
        for t, j in enumerate([0, 2, 3, 4, 5, 6, 7]):
            cols = pl.ds(j * HD, HD)
            _kv_rdma(pl.ds(0, KV1), cols, r1, t, 2, j).wait_send()
            _kv_rdma_v(pl.ds(0, KV1), cols, r1, 7 + t, 3, j).wait_send()

    for o in range(1, N):
        pltpu.make_async_remote_copy(
            src_ref=acc.at[pl.ds(0, CH), :], dst_ref=arbuf.at[o - 1],
            send_sem=ar_send.at[o - 1], recv_sem=ar_recv.at[o - 1],
            device_id=(0,), device_id_type=pl.DeviceIdType.MESH,
        ).wait_send()
        pltpu.make_async_remote_copy(
            src_ref=red, dst_ref=agbuf.at[o - 1],
            send_sem=ag_send.at[o - 1], recv_sem=ag_recv.at[o - 1],
            device_id=(0,), device_id_type=pl.DeviceIdType.MESH,
        ).wait_send()


def kernel(x, Wq, K_ext, V_ext, Wo):
    x16 = x.reshape(SQ, 1024).astype(jnp.bfloat16)
    wq16 = Wq.astype(jnp.bfloat16)
    wo16 = Wo.astype(jnp.bfloat16)
    k16 = K_ext.reshape(KV0, N * HD).astype(jnp.bfloat16)
    v16 = V_ext.reshape(KV0, N * HD).astype(jnp.bfloat16)

    out = pl.pallas_call(
        _body,
        out_shape=jax.ShapeDtypeStruct((SQ, 1024), jnp.float32),
        in_specs=[
            pl.BlockSpec(memory_space=pltpu.MemorySpace.VMEM),
            pl.BlockSpec(memory_space=pltpu.MemorySpace.VMEM),
            pl.BlockSpec(memory_space=pltpu.MemorySpace.HBM),
            pl.BlockSpec(memory_space=pltpu.MemorySpace.HBM),
            pl.BlockSpec(memory_space=pltpu.MemorySpace.VMEM),
        ],
        out_specs=pl.BlockSpec(memory_space=pltpu.MemorySpace.VMEM),
        scratch_shapes=[
            pltpu.VMEM((SQ, HD), jnp.bfloat16),
            pltpu.VMEM((KVW, HD), jnp.bfloat16),
            pltpu.VMEM((KVW, HD), jnp.bfloat16),
            pltpu.VMEM((SQ, HD), jnp.bfloat16),
            pltpu.VMEM((SQ, 1024), jnp.float32),
            pltpu.VMEM((CH, 1024), jnp.float32),
            pltpu.VMEM((N - 1, CH, 1024), jnp.float32),
            pltpu.VMEM((N - 1, CH, 1024), jnp.float32),
            pltpu.SemaphoreType.DMA((16,)),
            pltpu.SemaphoreType.DMA((4,)),
            pltpu.SemaphoreType.DMA((2,)),
            pltpu.SemaphoreType.DMA((N - 1,)),
            pltpu.SemaphoreType.DMA((N - 1,)),
            pltpu.SemaphoreType.DMA((N - 1,)),
            pltpu.SemaphoreType.DMA((N - 1,)),
        ],
        compiler_params=pltpu.CompilerParams(
            vmem_limit_bytes=100 * 1024 * 1024,
        ),
    )(x16, wq16, k16, v16, wo16)
    return out.reshape(1, SQ, 1024)


# device time: 296160 ns/iter; 1.2612x vs baseline; 1.2612x over previous
import functools

import jax
import jax.numpy as jnp
from jax import lax
from jax.experimental import pallas as pl
from jax.experimental.pallas import tpu as pltpu

N = 8
SQ = 1024
HP = 8
DH = 128
HD = HP * DH
KVW = 1152
KV0 = 1024
KV1 = 128
CH = SQ // N
SCALE = 0.08838834764831843

_RELAYS = {
    1: [("K", 2), ("V", 2)],
    3: [("K", 7), ("V", 7), ("K", 6)],
    4: [("K", 5), ("V", 5), ("V", 6)],
}
_DIRECT = [1, 3, 4]


def _body(x_ref, wq_ref, k_hbm, v_hbm, wo_ref, out_ref,
          q16, kbuf, vbuf, ctx16, acc, acc16, red, red16,
          arbuf, agbuf, relay_buf,
          kv_send, kv_recv, loc_sem,
          relay_send, relay_recv,
          ar_send, ar_recv, ag_send, ag_recv):
    my = lax.axis_index("i")

    def _kv_rdma(src_rows, src_cols, dst_rows, send_slot, recv_slot, dest):
        return pltpu.make_async_remote_copy(
            src_ref=k_hbm.at[src_rows, src_cols],
            dst_ref=kbuf.at[dst_rows, :],
            send_sem=kv_send.at[send_slot],
            recv_sem=kv_recv.at[recv_slot],
            device_id=(dest,),
            device_id_type=pl.DeviceIdType.MESH,
        )

    def _kv_rdma_v(src_rows, src_cols, dst_rows, send_slot, recv_slot, dest):
        return pltpu.make_async_remote_copy(
            src_ref=v_hbm.at[src_rows, src_cols],
            dst_ref=vbuf.at[dst_rows, :],
            send_sem=kv_send.at[send_slot],
            recv_sem=kv_recv.at[recv_slot],
            device_id=(dest,),
            device_id_type=pl.DeviceIdType.MESH,
        )

    r0 = pl.ds(0, KV0)
    r1 = pl.ds(KV0, KV1)

    def _relay_in(tensor, dest, idx, rpos, slot):
        src = (k_hbm if tensor == "K" else v_hbm)
        return pltpu.make_async_remote_copy(
            src_ref=src.at[pl.ds(0, KV0), pl.ds(dest * HD, HD)],
            dst_ref=relay_buf.at[idx],
            send_sem=kv_send.at[slot],
            recv_sem=relay_recv.at[idx],
            device_id=(rpos,),
            device_id_type=pl.DeviceIdType.MESH,
        )

    @pl.when(my == 0)
    def _():
        slot = 6
        for rpos, plist in _RELAYS.items():
            for idx, (tensor, dest) in enumerate(plist):
                _relay_in(tensor, dest, idx, rpos, slot).start()
                slot += 1
        for t, j in enumerate(_DIRECT):
            cols = pl.ds(j * HD, HD)
            _kv_rdma(pl.ds(0, KV0), cols, r0, 2 * t, 0, j).start()
            _kv_rdma_v(pl.ds(0, KV0), cols, r0, 2 * t + 1, 1, j).start()
        ck = pltpu.make_async_copy(
            k_hbm.at[pl.ds(0, KV0), pl.ds(0, HD)], kbuf.at[r0, :], loc_sem.at[0])
        cv = pltpu.make_async_copy(
            v_hbm.at[pl.ds(0, KV0), pl.ds(0, HD)], vbuf.at[r0, :], loc_sem.at[1])
        ck.start(); cv.start(); ck.wait(); cv.wait()

    @pl.when(my == 1)
    def _():
        for t, j in enumerate([0, 2, 3, 4, 5, 6, 7]):
            cols = pl.ds(j * HD, HD)
            _kv_rdma(pl.ds(0, KV1), cols, r1, t, 2, j).start()
            _kv_rdma_v(pl.ds(0, KV1), cols, r1, 7 + t, 3, j).start()
        ck = pltpu.make_async_copy(
            k_hbm.at[pl.ds(0, KV1), pl.ds(HD, HD)], kbuf.at[r1, :], loc_sem.at[0])
        cv = pltpu.make_async_copy(
            v_hbm.at[pl.ds(0, KV1), pl.ds(HD, HD)], vbuf.at[r1, :], loc_sem.at[1])
        ck.start(); cv.start(); ck.wait(); cv.wait()

    q = jnp.dot(x_ref[...], wq_ref[...], preferred_element_type=jnp.float32)
    q16[...] = (q * SCALE).astype(jnp.bfloat16)

    for rpos, plist in _RELAYS.items():
        @pl.when(my == rpos)
        def _(plist=plist):
            for idx, (tensor, dest) in enumerate(plist):
                pltpu.make_async_remote_copy(
                    src_ref=relay_buf.at[idx], dst_ref=relay_buf.at[idx],
                    send_sem=kv_send.at[15], recv_sem=relay_recv.at[idx],
                    device_id=(0,), device_id_type=pl.DeviceIdType.MESH,
                ).wait_recv()
                dstbuf = kbuf if tensor == "K" else vbuf
                pltpu.make_async_remote_copy(
                    src_ref=relay_buf.at[idx],
                    dst_ref=dstbuf.at[r0, :],
                    send_sem=relay_send.at[idx],
                    recv_sem=kv_recv.at[0 if tensor == "K" else 1],
                    device_id=(dest,),
                    device_id_type=pl.DeviceIdType.MESH,
                ).start()

    @pl.when(my != 0)
    def _():
        _kv_rdma(pl.ds(0, KV0), pl.ds(0, HD), r0, 15, 0, 0).wait_recv()
        _kv_rdma_v(pl.ds(0, KV0), pl.ds(0, HD), r0, 15, 1, 0).wait_recv()

    @pl.when(my != 1)
    def _():
        _kv_rdma(pl.ds(0, KV1), pl.ds(0, HD), r1, 15, 2, 1).wait_recv()
        _kv_rdma_v(pl.ds(0, KV1), pl.ds(0, HD), r1, 15, 3, 1).wait_recv()

    qi = lax.broadcasted_iota(jnp.int32, (SQ, KVW), 0)
    ki = lax.broadcasted_iota(jnp.int32, (SQ, KVW), 1)
    mask = jnp.abs(qi - ki) <= 128
    for h in range(HP):
        c = pl.ds(h * DH, DH)
        s = lax.dot_general(
            q16[:, c], kbuf[:, c],
            (((1,), (1,)), ((), ())), preferred_element_type=jnp.float32)
        s = jnp.where(mask, s, -1e9)
        m = jnp.max(s, axis=1, keepdims=True)
        w = jnp.exp(s - m)
        w = w / jnp.sum(w, axis=1, keepdims=True)
        ctx = lax.dot_general(
            w.astype(jnp.bfloat16), vbuf[:, c],
            (((1,), (0,)), ((), ())), preferred_element_type=jnp.float32)
        ctx16[:, c] = ctx.astype(jnp.bfloat16)

    acc[...] = jnp.dot(ctx16[...], wo_ref[...],
                       preferred_element_type=jnp.float32)
    acc16[...] = acc[...].astype(jnp.bfloat16)

    for o in range(1, N):
        dest = lax.rem(my + o, N)
        pltpu.make_async_remote_copy(
            src_ref=acc16.at[pl.ds(dest * CH, CH), :],
            dst_ref=arbuf.at[o - 1],
            send_sem=ar_send.at[o - 1],
            recv_sem=ar_recv.at[o - 1],
            device_id=(dest,),
            device_id_type=pl.DeviceIdType.MESH,
        ).start()
    for o in range(1, N):
        pltpu.make_async_remote_copy(
            src_ref=arbuf.at[o - 1], dst_ref=arbuf.at[o - 1],
            send_sem=ar_send.at[o - 1], recv_sem=ar_recv.at[o - 1],
            device_id=(0,), device_id_type=pl.DeviceIdType.MESH,
        ).wait_recv()

    r = acc[pl.ds(my * CH, CH), :]
    for t in range(N - 1):
        r = r + arbuf[t].astype(jnp.float32)
    red[...] = r
    red16[...] = r.astype(jnp.bfloat16)
    out_ref[pl.ds(my * CH, CH), :] = r

    for o in range(1, N):
        dest = lax.rem(my + o, N)
        pltpu.make_async_remote_copy(
            src_ref=red16,
            dst_ref=agbuf.at[o - 1],
            send_sem=ag_send.at[o - 1],
            recv_sem=ag_recv.at[o - 1],
            device_id=(dest,),
            device_id_type=pl.DeviceIdType.MESH,
        ).start()
    for o in range(1, N):
        pltpu.make_async_remote_copy(
            src_ref=agbuf.at[o - 1], dst_ref=agbuf.at[o - 1],
            send_sem=ag_send.at[o - 1], recv_sem=ag_recv.at[o - 1],
            device_id=(0,), device_id_type=pl.DeviceIdType.MESH,
        ).wait_recv()
        src_pos = lax.rem(my - o + N, N)
        out_ref[pl.ds(src_pos * CH, CH), :] = agbuf[o - 1].astype(jnp.float32)

    @pl.when(my == 0)
    def _():
        slot = 6
        for rpos, plist in _RELAYS.items():
            for idx, (tensor, dest) in enumerate(plist):
                _relay_in(tensor, dest, idx, rpos, slot).wait_send()
                slot += 1
        for t, j in enumerate(_DIRECT):
            cols = pl.ds(j * HD, HD)
            _kv_rdma(pl.ds(0, KV0), cols, r0, 2 * t, 0, j).wait_send()
            _kv_rdma_v(pl.ds(0, KV0), cols, r0, 2 * t + 1, 1, j).wait_send()

    @pl.when(my == 1)
    def _():
        for t, j in enumerate([0, 2, 3, 4, 5, 6, 7]):
            cols = pl.ds(j * HD, HD)
            _kv_rdma(pl.ds(0, KV1), cols, r1, t, 2, j).wait_send()
            _kv_rdma_v(pl.ds(0, KV1), cols, r1, 7 + t, 3, j).wait_send()

    for rpos, plist in _RELAYS.items():
        @pl.when(my == rpos)
        def _(plist=plist):
            for idx, (tensor, dest) in enumerate(plist):
                dstbuf = kbuf if tensor == "K" else vbuf
                pltpu.make_async_remote_copy(
                    src_ref=relay_buf.at[idx], dst_ref=dstbuf.at[r0, :],
                    send_sem=relay_send.at[idx],
                    recv_sem=kv_recv.at[0 if tensor == "K" else 1],
                    device_id=(dest,), device_id_type=pl.DeviceIdType.MESH,
                ).wait_send()

    for o in range(1, N):
        pltpu.make_async_remote_copy(
            src_ref=acc16.at[pl.ds(0, CH), :], dst_ref=arbuf.at[o - 1],
            send_sem=ar_send.at[o - 1], recv_sem=ar_recv.at[o - 1],
            device_id=(0,), device_id_type=pl.DeviceIdType.MESH,
        ).wait_send()
        pltpu.make_async_remote_copy(
            src_ref=red16, dst_ref=agbuf.at[o - 1],
            send_sem=ag_send.at[o - 1], recv_sem=ag_recv.at[o - 1],
            device_id=(0,), device_id_type=pl.DeviceIdType.MESH,
        ).wait_send()


def kernel(x, Wq, K_ext, V_ext, Wo):
    x16 = x.reshape(SQ, 1024).astype(jnp.bfloat16)
    wq16 = Wq.astype(jnp.bfloat16)
    wo16 = Wo.astype(jnp.bfloat16)
    k16 = K_ext.reshape(KV0, N * HD).astype(jnp.bfloat16)
    v16 = V_ext.reshape(KV0, N * HD).astype(jnp.bfloat16)

    out = pl.pallas_call(
        _body,
        out_shape=jax.ShapeDtypeStruct((SQ, 1024), jnp.float32),
        in_specs=[
            pl.BlockSpec(memory_space=pltpu.MemorySpace.VMEM),
            pl.BlockSpec(memory_space=pltpu.MemorySpace.VMEM),
            pl.BlockSpec(memory_space=pltpu.MemorySpace.HBM),
            pl.BlockSpec(memory_space=pltpu.MemorySpace.HBM),
            pl.BlockSpec(memory_space=pltpu.MemorySpace.VMEM),
        ],
        out_specs=pl.BlockSpec(memory_space=pltpu.MemorySpace.VMEM),
        scratch_shapes=[
            pltpu.VMEM((SQ, HD), jnp.bfloat16),
            pltpu.VMEM((KVW, HD), jnp.bfloat16),
            pltpu.VMEM((KVW, HD), jnp.bfloat16),
            pltpu.VMEM((SQ, HD), jnp.bfloat16),
            pltpu.VMEM((SQ, 1024), jnp.float32),
            pltpu.VMEM((SQ, 1024), jnp.bfloat16),
            pltpu.VMEM((CH, 1024), jnp.float32),
            pltpu.VMEM((CH, 1024), jnp.bfloat16),
            pltpu.VMEM((N - 1, CH, 1024), jnp.bfloat16),
            pltpu.VMEM((N - 1, CH, 1024), jnp.bfloat16),
            pltpu.VMEM((3, KV0, HD), jnp.bfloat16),
            pltpu.SemaphoreType.DMA((16,)),
            pltpu.SemaphoreType.DMA((4,)),
            pltpu.SemaphoreType.DMA((2,)),
            pltpu.SemaphoreType.DMA((3,)),
            pltpu.SemaphoreType.DMA((3,)),
            pltpu.SemaphoreType.DMA((N - 1,)),
            pltpu.SemaphoreType.DMA((N - 1,)),
            pltpu.SemaphoreType.DMA((N - 1,)),
            pltpu.SemaphoreType.DMA((N - 1,)),
        ],
        compiler_params=pltpu.CompilerParams(
            vmem_limit_bytes=100 * 1024 * 1024,
        ),
    )(x16, wq16, k16, v16, wo16)
    return out.reshape(1, SQ, 1024)


# device time: 274537 ns/iter; 1.3606x vs baseline; 1.0788x over previous
import jax
import jax.numpy as jnp
from jax import lax
from jax.experimental import pallas as pl
from jax.experimental.pallas import tpu as pltpu

N = 8
SQ = 1024
HP = 8
DH = 128
HD = HP * DH
KVW = 1152
KV0 = 1024
KV1 = 128
CH = SQ // N
SCALE = 0.08838834764831843

_LINK_PLANS = [
    [("K", 2, (1, 0)), ("V", 2, (1, 1)), ("K", 1, None), ("V", 1, None)],
    [("K", 7, (3, 0)), ("V", 7, (3, 1)), ("K", 6, (3, 2)),
     ("K", 3, None), ("V", 3, None)],
    [("K", 5, (4, 0)), ("V", 5, (4, 1)), ("V", 6, (4, 2)),
     ("K", 4, None), ("V", 4, None)],
]
_RELAYS = {
    1: [("K", 2), ("V", 2)],
    3: [("K", 7), ("V", 7), ("K", 6)],
    4: [("K", 5), ("V", 5), ("V", 6)],
}


def _round_robin(plans):
    out = []
    r = 0
    while any(r < len(p) for p in plans):
        for li, p in enumerate(plans):
            if r < len(p):
                out.append((li, r, p[r]))
        r += 1
    return out


def _body(x_ref, wq_ref, k_hbm, v_hbm, wo_ref, out_ref,
          q16, kbuf, vbuf, ctx16, acc16, red, red16,
          arbuf, agbuf, relay_buf, stage16, stageF,
          kv_send, kv_recv, loc_sem,
          relay_send, relay_recv,
          ar_send, ar_recv, ag_send, ag_recv):
    my = lax.axis_index("i")

    r0 = pl.ds(0, KV0)
    r1 = pl.ds(KV0, KV1)

    def _dst(tensor, rows):
        return (kbuf if tensor == "K" else vbuf).at[rows, :]

    def _stage_send(rows_len, payloads, src_row0):
        prev = [None] * 6
        counters = [0, 0, 0]
        rows = pl.ds(0, rows_len)
        for li, _, (tensor, dest, relay) in payloads:
            slot = 2 * li + (counters[li] % 2)
            counters[li] += 1
            if prev[slot] is not None:
                prev[slot].wait_send()
            hbm = k_hbm if tensor == "K" else v_hbm
            cp = pltpu.make_async_copy(
                hbm.at[pl.ds(src_row0, rows_len), pl.ds(dest * HD, HD)],
                stageF.at[rows, :], loc_sem.at[0])
            cp.start()
            cp.wait()
            stage16[slot, rows, :] = stageF[rows, :].astype(jnp.bfloat16)
            if relay is None:
                dst = _dst(tensor, r0 if rows_len == KV0 else r1)
                rsem = kv_recv.at[(0 if tensor == "K" else 1)
                                  + (0 if rows_len == KV0 else 2)]
            else:
                rpos, ridx = relay
                dst = relay_buf.at[ridx, rows, :] if rows_len == KV0 \
                    else relay_buf.at[ridx]
                rsem = relay_recv.at[ridx]
                dest = rpos
            rd = pltpu.make_async_remote_copy(
                src_ref=stage16.at[slot, rows, :],
                dst_ref=dst,
                send_sem=kv_send.at[slot],
                recv_sem=rsem,
                device_id=(dest,),
                device_id_type=pl.DeviceIdType.MESH,
            )
            rd.start()
            prev[slot] = rd
        return prev

    @pl.when(my == 0)
    def _():
        prev = _stage_send(KV0, _round_robin(_LINK_PLANS), 0)
        for tensor in ("K", "V"):
            hbm = k_hbm if tensor == "K" else v_hbm
            cp = pltpu.make_async_copy(
                hbm.at[pl.ds(0, KV0), pl.ds(0, HD)],
                stageF.at[pl.ds(0, KV0), :], loc_sem.at[0])
            cp.start()
            cp.wait()
            b = kbuf if tensor == "K" else vbuf
            b[r0, :] = stageF[pl.ds(0, KV0), :].astype(jnp.bfloat16)
        for d in prev:
            if d is not None:
                d.wait_send()

    @pl.when(my == 1)
    def _():
        plans1 = [[("K", j, None), ("V", j, None)]
                  for j in (0, 2, 3, 4, 5, 6, 7)]
        flat = [e for p in plans1 for e in p]
        plans1 = [flat[0:5], flat[5:10], flat[10:14]]
        prev = _stage_send(KV1, _round_robin(plans1), 0)
        for tensor in ("K", "V"):
            hbm = k_hbm if tensor == "K" else v_hbm
            cp = pltpu.make_async_copy(
                hbm.at[pl.ds(0, KV1), pl.ds(HD, HD)],
                stageF.at[pl.ds(0, KV1), :], loc_sem.at[0])
            cp.start()
            cp.wait()
            b = kbuf if tensor == "K" else vbuf
            b[r1, :] = stageF[pl.ds(0, KV1), :].astype(jnp.bfloat16)
        for d in prev:
            if d is not None:
                d.wait_send()

    q = jnp.dot(x_ref[...].astype(jnp.bfloat16),
                wq_ref[...].astype(jnp.bfloat16),
                preferred_element_type=jnp.float32)
    q16[...] = (q * SCALE).astype(jnp.bfloat16)

    for rpos, plist in _RELAYS.items():
        @pl.when(my == rpos)
        def _(plist=plist):
            for idx, (tensor, dest) in enumerate(plist):
                pltpu.make_async_remote_copy(
                    src_ref=relay_buf.at[idx], dst_ref=relay_buf.at[idx],
                    send_sem=kv_send.at[0], recv_sem=relay_recv.at[idx],
                    device_id=(0,), device_id_type=pl.DeviceIdType.MESH,
                ).wait_recv()
                pltpu.make_async_remote_copy(
                    src_ref=relay_buf.at[idx],
                    dst_ref=_dst(tensor, r0),
                    send_sem=relay_send.at[idx],
                    recv_sem=kv_recv.at[0 if tensor == "K" else 1],
                    device_id=(dest,),
                    device_id_type=pl.DeviceIdType.MESH,
                ).start()

    def _kv_wait(rows, slot):
        pltpu.make_async_remote_copy(
            src_ref=kbuf.at[rows, :] if slot % 2 == 0 else vbuf.at[rows, :],
            dst_ref=kbuf.at[rows, :] if slot % 2 == 0 else vbuf.at[rows, :],
            send_sem=kv_send.at[0], recv_sem=kv_recv.at[slot],
            device_id=(0,), device_id_type=pl.DeviceIdType.MESH,
        ).wait_recv()

    @pl.when(my != 0)
    def _():
        _kv_wait(r0, 0)
        _kv_wait(r0, 1)

    @pl.when(my != 1)
    def _():
        _kv_wait(r1, 2)
        _kv_wait(r1, 3)

    qi = lax.broadcasted_iota(jnp.int32, (SQ, KVW), 0)
    ki = lax.broadcasted_iota(jnp.int32, (SQ, KVW), 1)
    mask = jnp.abs(qi - ki) <= 128
    for h in range(HP):
        c = pl.ds(h * DH, DH)
        s = lax.dot_general(
            q16[:, c], kbuf[:, c],
            (((1,), (1,)), ((), ())), preferred_element_type=jnp.float32)
        s = jnp.where(mask, s, -1e9)
        m = jnp.max(s, axis=1, keepdims=True)
        w = jnp.exp(s - m)
        w = w / jnp.sum(w, axis=1, keepdims=True)
        ctx = lax.dot_general(
            w.astype(jnp.bfloat16), vbuf[:, c],
            (((1,), (0,)), ((), ())), preferred_element_type=jnp.float32)
        ctx16[:, c] = ctx.astype(jnp.bfloat16)

    acc = jnp.dot(ctx16[...], wo_ref[...].astype(jnp.bfloat16),
                  preferred_element_type=jnp.float32)
    acc16[...] = acc.astype(jnp.bfloat16)

    for o in range(1, N):
        dest = lax.rem(my + o, N)
        pltpu.make_async_remote_copy(
            src_ref=acc16.at[pl.ds(dest * CH, CH), :],
            dst_ref=arbuf.at[o - 1],
            send_sem=ar_send.at[o - 1],
            recv_sem=ar_recv.at[o - 1],
            device_id=(dest,),
            device_id_type=pl.DeviceIdType.MESH,
        ).start()
    for o in range(1, N):
        pltpu.make_async_remote_copy(
            src_ref=arbuf.at[o - 1], dst_ref=arbuf.at[o - 1],
            send_sem=ar_send.at[o - 1], recv_sem=ar_recv.at[o - 1],
            device_id=(0,), device_id_type=pl.DeviceIdType.MESH,
        ).wait_recv()

    r = acc16[pl.ds(my * CH, CH), :].astype(jnp.float32)
    for t in range(N - 1):
        r = r + arbuf[t].astype(jnp.float32)
    red[...] = r
    red16[...] = r.astype(jnp.bfloat16)
    out_ref[pl.ds(my * CH, CH), :] = r

    for o in range(1, N):
        dest = lax.rem(my + o, N)
        pltpu.make_async_remote_copy(
            src_ref=red16,
            dst_ref=agbuf.at[o - 1],
            send_sem=ag_send.at[o - 1],
            recv_sem=ag_recv.at[o - 1],
            device_id=(dest,),
            device_id_type=pl.DeviceIdType.MESH,
        ).start()
    for o in range(1, N):
        pltpu.make_async_remote_copy(
            src_ref=agbuf.at[o - 1], dst_ref=agbuf.at[o - 1],
            send_sem=ag_send.at[o - 1], recv_sem=ag_recv.at[o - 1],
            device_id=(0,), device_id_type=pl.DeviceIdType.MESH,
        ).wait_recv()
        src_pos = lax.rem(my - o + N, N)
        out_ref[pl.ds(src_pos * CH, CH), :] = agbuf[o - 1].astype(jnp.float32)

    for rpos, plist in _RELAYS.items():
        @pl.when(my == rpos)
        def _(plist=plist):
            for idx, (tensor, dest) in enumerate(plist):
                pltpu.make_async_remote_copy(
                    src_ref=relay_buf.at[idx], dst_ref=_dst(tensor, r0),
                    send_sem=relay_send.at[idx],
                    recv_sem=kv_recv.at[0 if tensor == "K" else 1],
                    device_id=(dest,), device_id_type=pl.DeviceIdType.MESH,
                ).wait_send()

    for o in range(1, N):
        pltpu.make_async_remote_copy(
            src_ref=acc16.at[pl.ds(0, CH), :], dst_ref=arbuf.at[o - 1],
            send_sem=ar_send.at[o - 1], recv_sem=ar_recv.at[o - 1],
            device_id=(0,), device_id_type=pl.DeviceIdType.MESH,
        ).wait_send()
        pltpu.make_async_remote_copy(
            src_ref=red16, dst_ref=agbuf.at[o - 1],
            send_sem=ag_send.at[o - 1], recv_sem=ag_recv.at[o - 1],
            device_id=(0,), device_id_type=pl.DeviceIdType.MESH,
        ).wait_send()


def kernel(x, Wq, K_ext, V_ext, Wo):
    out = pl.pallas_call(
        _body,
        out_shape=jax.ShapeDtypeStruct((SQ, 1024), jnp.float32),
        in_specs=[
            pl.BlockSpec(memory_space=pltpu.MemorySpace.VMEM),
            pl.BlockSpec(memory_space=pltpu.MemorySpace.VMEM),
            pl.BlockSpec(memory_space=pltpu.MemorySpace.HBM),
            pl.BlockSpec(memory_space=pltpu.MemorySpace.HBM),
            pl.BlockSpec(memory_space=pltpu.MemorySpace.VMEM),
        ],
        out_specs=pl.BlockSpec(memory_space=pltpu.MemorySpace.VMEM),
        scratch_shapes=[
            pltpu.VMEM((SQ, HD), jnp.bfloat16),
            pltpu.VMEM((KVW, HD), jnp.bfloat16),
            pltpu.VMEM((KVW, HD), jnp.bfloat16),
            pltpu.VMEM((SQ, HD), jnp.bfloat16),
            pltpu.VMEM((SQ, 1024), jnp.bfloat16),
            pltpu.VMEM((CH, 1024), jnp.float32),
            pltpu.VMEM((CH, 1024), jnp.bfloat16),
            pltpu.VMEM((N - 1, CH, 1024), jnp.bfloat16),
            pltpu.VMEM((N - 1, CH, 1024), jnp.bfloat16),
            pltpu.VMEM((3, KV0, HD), jnp.bfloat16),
            pltpu.VMEM((6, KV0, HD), jnp.bfloat16),
            pltpu.VMEM((KV0, HD), jnp.float32),
            pltpu.SemaphoreType.DMA((6,)),
            pltpu.SemaphoreType.DMA((4,)),
            pltpu.SemaphoreType.DMA((1,)),
            pltpu.SemaphoreType.DMA((3,)),
            pltpu.SemaphoreType.DMA((3,)),
            pltpu.SemaphoreType.DMA((N - 1,)),
            pltpu.SemaphoreType.DMA((N - 1,)),
            pltpu.SemaphoreType.DMA((N - 1,)),
            pltpu.SemaphoreType.DMA((N - 1,)),
        ],
        compiler_params=pltpu.CompilerParams(
            vmem_limit_bytes=100 * 1024 * 1024,
        ),
    )(x.reshape(SQ, 1024), Wq, K_ext.reshape(KV0, N * HD),
      V_ext.reshape(KV0, N * HD), Wo)
    return out.reshape(1, SQ, 1024)


# device time: 197276 ns/iter; 1.8934x vs baseline; 1.3916x over previous
import jax
import jax.numpy as jnp
from jax import lax
from jax.experimental import pallas as pl
from jax.experimental.pallas import tpu as pltpu

N = 8
SQ = 1024
HP = 8
DH = 128
HD = HP * DH
KVW = 1152
KV0 = 1024
KV1 = 128
CH = SQ // N
SCALE = 0.08838834764831843

_LINK_PLANS = [
    [("K", 2, (1, 0)), ("V", 2, (1, 1)), ("K", 1, None), ("V", 1, None)],
    [("K", 7, (3, 0)), ("V", 7, (3, 1)), ("K", 6, (3, 2)),
     ("K", 3, None), ("V", 3, None)],
    [("K", 5, (4, 0)), ("V", 5, (4, 1)), ("V", 6, (4, 2)),
     ("K", 4, None), ("V", 4, None)],
]
_RELAYS = {
    1: [("K", 2), ("V", 2)],
    3: [("K", 7), ("V", 7), ("K", 6)],
    4: [("K", 5), ("V", 5), ("V", 6)],
}


def _round_robin(plans):
    out = []
    r = 0
    while any(r < len(p) for p in plans):
        for li, p in enumerate(plans):
            if r < len(p):
                out.append((li, p[r]))
        r += 1
    return out


def _body(x_ref, wq_ref, k_hbm, v_hbm, wo_ref, out_ref,
          q16, kbuf, vbuf, ctx16, acc16, red, red16,
          arbuf, agbuf, relay_buf, stage16, stageF,
          kv_send, kv_recv, loc_sem,
          relay_send, relay_recv,
          ar_send, ar_recv, ag_send, ag_recv):
    my = lax.axis_index("i")

    r0 = pl.ds(0, KV0)
    r1 = pl.ds(KV0, KV1)

    def _dst(tensor, rows):
        return (kbuf if tensor == "K" else vbuf).at[:, rows, :]

    def _pull_heads(tensor, head0, n_rows, dst):
        hbm = k_hbm if tensor == "K" else v_hbm
        cps = []
        for h in range(HP):
            cp = pltpu.make_async_copy(
                hbm.at[0, pl.ds(0, n_rows), head0 + h, :],
                dst.at[h, pl.ds(0, n_rows), :],
                loc_sem.at[h])
            cp.start()
            cps.append(cp)
        for cp in cps:
            cp.wait()

    def _stage_send(n_rows, payloads):
        prev = [None] * 6
        counters = [0, 0, 0]
        rows = pl.ds(0, n_rows)
        for li, (tensor, dest, relay) in payloads:
            slot = 2 * li + (counters[li] % 2)
            counters[li] += 1
            if prev[slot] is not None:
                prev[slot].wait_send()
            _pull_heads(tensor, dest * HP, n_rows, stageF)
            stage16[slot, :, rows, :] = stageF[:, rows, :].astype(jnp.bfloat16)
            if relay is None:
                dst = _dst(tensor, r0 if n_rows == KV0 else r1)
                rsem = kv_recv.at[(0 if tensor == "K" else 1)
                                  + (0 if n_rows == KV0 else 2)]
            else:
                rpos, ridx = relay
                dst = relay_buf.at[ridx]
                rsem = relay_recv.at[ridx]
                dest = rpos
            rd = pltpu.make_async_remote_copy(
                src_ref=stage16.at[slot, :, rows, :],
                dst_ref=dst,
                send_sem=kv_send.at[slot],
                recv_sem=rsem,
                device_id=(dest,),
                device_id_type=pl.DeviceIdType.MESH,
            )
            rd.start()
            prev[slot] = rd
        return prev

    @pl.when(my == 0)
    def _():
        prev = _stage_send(KV0, _round_robin(_LINK_PLANS))
        for tensor in ("K", "V"):
            _pull_heads(tensor, 0, KV0, stageF)
            b = kbuf if tensor == "K" else vbuf
            b[:, r0, :] = stageF[...].astype(jnp.bfloat16)
        for d in prev:
            if d is not None:
                d.wait_send()

    @pl.when(my == 1)
    def _():
        flat = []
        for j in (0, 2, 3, 4, 5, 6, 7):
            flat += [("K", j, None), ("V", j, None)]
        plans1 = [flat[0:5], flat[5:10], flat[10:14]]
        prev = _stage_send(KV1, _round_robin(plans1))
        for tensor in ("K", "V"):
            _pull_heads(tensor, HP, KV1, stageF)
            b = kbuf if tensor == "K" else vbuf
            b[:, r1, :] = stageF[:, pl.ds(0, KV1), :].astype(jnp.bfloat16)
        for d in prev:
            if d is not None:
                d.wait_send()

    q = jnp.dot(x_ref[...].astype(jnp.bfloat16),
                wq_ref[...].astype(jnp.bfloat16),
                preferred_element_type=jnp.float32)
    q16[...] = (q * SCALE).astype(jnp.bfloat16)

    for rpos, plist in _RELAYS.items():
        @pl.when(my == rpos)
        def _(plist=plist):
            for idx, (tensor, dest) in enumerate(plist):
                pltpu.make_async_remote_copy(
                    src_ref=relay_buf.at[idx], dst_ref=relay_buf.at[idx],
                    send_sem=kv_send.at[0], recv_sem=relay_recv.at[idx],
                    device_id=(0,), device_id_type=pl.DeviceIdType.MESH,
                ).wait_recv()
                pltpu.make_async_remote_copy(
                    src_ref=relay_buf.at[idx],
                    dst_ref=_dst(tensor, r0),
                    send_sem=relay_send.at[idx],
                    recv_sem=kv_recv.at[0 if tensor == "K" else 1],
                    device_id=(dest,),
                    device_id_type=pl.DeviceIdType.MESH,
                ).start()

    def _kv_wait(rows, slot):
        b = kbuf if slot % 2 == 0 else vbuf
        pltpu.make_async_remote_copy(
            src_ref=b.at[:, rows, :], dst_ref=b.at[:, rows, :],
            send_sem=kv_send.at[0], recv_sem=kv_recv.at[slot],
            device_id=(0,), device_id_type=pl.DeviceIdType.MESH,
        ).wait_recv()

    @pl.when(my != 0)
    def _():
        _kv_wait(r0, 0)
        _kv_wait(r0, 1)

    @pl.when(my != 1)
    def _():
        _kv_wait(r1, 2)
        _kv_wait(r1, 3)

    qi = lax.broadcasted_iota(jnp.int32, (SQ, KVW), 0)
    ki = lax.broadcasted_iota(jnp.int32, (SQ, KVW), 1)
    mask = jnp.abs(qi - ki) <= 128
    for h in range(HP):
        c = pl.ds(h * DH, DH)
        s = lax.dot_general(
            q16[:, c], kbuf[h],
            (((1,), (1,)), ((), ())), preferred_element_type=jnp.float32)
        s = jnp.where(mask, s, -1e9)
        m = jnp.max(s, axis=1, keepdims=True)
        w = jnp.exp(s - m)
        w = w / jnp.sum(w, axis=1, keepdims=True)
        ctx = lax.dot_general(
            w.astype(jnp.bfloat16), vbuf[h],
            (((1,), (0,)), ((), ())), preferred_element_type=jnp.float32)
        ctx16[:, c] = ctx.astype(jnp.bfloat16)

    acc = jnp.dot(ctx16[...], wo_ref[...].astype(jnp.bfloat16),
                  preferred_element_type=jnp.float32)
    acc16[...] = acc.astype(jnp.bfloat16)

    for o in range(1, N):
        dest = lax.rem(my + o, N)
        pltpu.make_async_remote_copy(
            src_ref=acc16.at[pl.ds(dest * CH, CH), :],
            dst_ref=arbuf.at[o - 1],
            send_sem=ar_send.at[o - 1],
            recv_sem=ar_recv.at[o - 1],
            device_id=(dest,),
            device_id_type=pl.DeviceIdType.MESH,
        ).start()
    for o in range(1, N):
        pltpu.make_async_remote_copy(
            src_ref=arbuf.at[o - 1], dst_ref=arbuf.at[o - 1],
            send_sem=ar_send.at[o - 1], recv_sem=ar_recv.at[o - 1],
            device_id=(0,), device_id_type=pl.DeviceIdType.MESH,
        ).wait_recv()

    r = acc16[pl.ds(my * CH, CH), :].astype(jnp.float32)
    for t in range(N - 1):
        r = r + arbuf[t].astype(jnp.float32)
    red[...] = r
    red16[...] = r.astype(jnp.bfloat16)
    out_ref[pl.ds(my * CH, CH), :] = r

    for o in range(1, N):
        dest = lax.rem(my + o, N)
        pltpu.make_async_remote_copy(
            src_ref=red16,
            dst_ref=agbuf.at[o - 1],
            send_sem=ag_send.at[o - 1],
            recv_sem=ag_recv.at[o - 1],
            device_id=(dest,),
            device_id_type=pl.DeviceIdType.MESH,
        ).start()
    for o in range(1, N):
        pltpu.make_async_remote_copy(
            src_ref=agbuf.at[o - 1], dst_ref=agbuf.at[o - 1],
            send_sem=ag_send.at[o - 1], recv_sem=ag_recv.at[o - 1],
            device_id=(0,), device_id_type=pl.DeviceIdType.MESH,
        ).wait_recv()
        src_pos = lax.rem(my - o + N, N)
        out_ref[pl.ds(src_pos * CH, CH), :] = agbuf[o - 1].astype(jnp.float32)

    for rpos, plist in _RELAYS.items():
        @pl.when(my == rpos)
        def _(plist=plist):
            for idx, (tensor, dest) in enumerate(plist):
                pltpu.make_async_remote_copy(
                    src_ref=relay_buf.at[idx], dst_ref=_dst(tensor, r0),
                    send_sem=relay_send.at[idx],
                    recv_sem=kv_recv.at[0 if tensor == "K" else 1],
                    device_id=(dest,), device_id_type=pl.DeviceIdType.MESH,
                ).wait_send()

    for o in range(1, N):
        pltpu.make_async_remote_copy(
            src_ref=acc16.at[pl.ds(0, CH), :], dst_ref=arbuf.at[o - 1],
            send_sem=ar_send.at[o - 1], recv_sem=ar_recv.at[o - 1],
            device_id=(0,), device_id_type=pl.DeviceIdType.MESH,
        ).wait_send()
        pltpu.make_async_remote_copy(
            src_ref=red16, dst_ref=agbuf.at[o - 1],
            send_sem=ag_send.at[o - 1], recv_sem=ag_recv.at[o - 1],
            device_id=(0,), device_id_type=pl.DeviceIdType.MESH,
        ).wait_send()


def kernel(x, Wq, K_ext, V_ext, Wo):
    out = pl.pallas_call(
        _body,
        out_shape=jax.ShapeDtypeStruct((SQ, 1024), jnp.float32),
        in_specs=[
            pl.BlockSpec(memory_space=pltpu.MemorySpace.VMEM),
            pl.BlockSpec(memory_space=pltpu.MemorySpace.VMEM),
            pl.BlockSpec(memory_space=pltpu.MemorySpace.HBM),
            pl.BlockSpec(memory_space=pltpu.MemorySpace.HBM),
            pl.BlockSpec(memory_space=pltpu.MemorySpace.VMEM),
        ],
        out_specs=pl.BlockSpec(memory_space=pltpu.MemorySpace.VMEM),
        scratch_shapes=[
            pltpu.VMEM((SQ, HD), jnp.bfloat16),
            pltpu.VMEM((HP, KVW, DH), jnp.bfloat16),
            pltpu.VMEM((HP, KVW, DH), jnp.bfloat16),
            pltpu.VMEM((SQ, HD), jnp.bfloat16),
            pltpu.VMEM((SQ, 1024), jnp.bfloat16),
            pltpu.VMEM((CH, 1024), jnp.float32),
            pltpu.VMEM((CH, 1024), jnp.bfloat16),
            pltpu.VMEM((N - 1, CH, 1024), jnp.bfloat16),
            pltpu.VMEM((N - 1, CH, 1024), jnp.bfloat16),
            pltpu.VMEM((3, HP, KV0, DH), jnp.bfloat16),
            pltpu.VMEM((6, HP, KV0, DH), jnp.bfloat16),
            pltpu.VMEM((HP, KV0, DH), jnp.float32),
            pltpu.SemaphoreType.DMA((6,)),
            pltpu.SemaphoreType.DMA((4,)),
            pltpu.SemaphoreType.DMA((HP,)),
            pltpu.SemaphoreType.DMA((3,)),
            pltpu.SemaphoreType.DMA((3,)),
            pltpu.SemaphoreType.DMA((N - 1,)),
            pltpu.SemaphoreType.DMA((N - 1,)),
            pltpu.SemaphoreType.DMA((N - 1,)),
            pltpu.SemaphoreType.DMA((N - 1,)),
        ],
        compiler_params=pltpu.CompilerParams(
            vmem_limit_bytes=100 * 1024 * 1024,
        ),
    )(x.reshape(SQ, 1024), Wq, K_ext, V_ext, Wo)
    return out.reshape(1, SQ, 1024)
